# Initial kernel scaffold; baseline (speedup 1.0000x reference)
import jax
import jax.numpy as jnp
from jax.experimental import pallas as pl


def kernel(words, weight):
    raise NotImplementedError("write your pallas kernel here")


# speedup vs baseline: 5.0599x; 5.0599x over previous
import functools

import jax
import jax.numpy as jnp
from jax import lax
from jax.experimental import pallas as pl
from jax.experimental.pallas import tpu as pltpu
from jax.experimental.pallas import tpu_sc as plsc

N_WORDS = 100000
N_LABELS = 128

BV = 5000
NB = N_WORDS // BV


def _negc_body(w_ref, out_ref, m_ref, s_ref):
    i = pl.program_id(0)

    @pl.when(i == 0)
    def _init():
        m_ref[...] = jnp.full_like(m_ref[...], -jnp.inf)
        s_ref[...] = jnp.zeros_like(s_ref[...])

    blk = w_ref[...]
    bm = jnp.max(blk, axis=0, keepdims=True)
    m_old = m_ref[...]
    m_new = jnp.maximum(m_old, bm)
    s_ref[...] = (s_ref[...] * jnp.exp(m_old - m_new)
                  + jnp.sum(jnp.exp(blk - m_new), axis=0, keepdims=True))
    m_ref[...] = m_new

    @pl.when(i == NB - 1)
    def _fin():
        out_ref[...] = -(m_ref[...] + jnp.log(s_ref[...]))


def _compute_negc(weight):
    return pl.pallas_call(
        _negc_body,
        grid=(NB,),
        in_specs=[pl.BlockSpec((BV, N_LABELS), lambda i: (i, 0))],
        out_specs=pl.BlockSpec((1, N_LABELS), lambda i: (0, 0)),
        out_shape=jax.ShapeDtypeStruct((1, N_LABELS), jnp.float32),
        scratch_shapes=[
            pltpu.VMEM((1, N_LABELS), jnp.float32),
            pltpu.VMEM((1, N_LABELS), jnp.float32),
        ],
        compiler_params=pltpu.CompilerParams(
            dimension_semantics=("arbitrary",)),
    )(weight)


def _logp_body(w_ref, negc_ref, out_ref):
    out_ref[...] = w_ref[...] + negc_ref[...]


def _compute_logp(weight, negc):
    return pl.pallas_call(
        _logp_body,
        grid=(NB,),
        in_specs=[
            pl.BlockSpec((BV, N_LABELS), lambda i: (i, 0)),
            pl.BlockSpec((1, N_LABELS), lambda i: (0, 0)),
        ],
        out_specs=pl.BlockSpec((BV, N_LABELS), lambda i: (i, 0)),
        out_shape=jax.ShapeDtypeStruct((N_WORDS, N_LABELS), jnp.float32),
        compiler_params=pltpu.CompilerParams(
            dimension_semantics=("parallel",)),
    )(weight, negc)


NC = 2
NS = 16
NW = NC * NS
TOK = 1024 * 200
CH = 128
B_PER_W = TOK // NW
NCH = B_PER_W // CH


@functools.partial(
    pl.kernel,
    mesh=plsc.VectorSubcoreMesh(core_axis_name="c", subcore_axis_name="s"),
    out_type=jax.ShapeDtypeStruct((TOK, N_LABELS), jnp.float32),
    scratch_types=[
        pltpu.VMEM((NCH, CH), jnp.int32),
        pltpu.VMEM((CH, N_LABELS), jnp.float32),
        pltpu.VMEM((CH, N_LABELS), jnp.float32),
        pltpu.SemaphoreType.DMA,
        pltpu.SemaphoreType.DMA,
        pltpu.SemaphoreType.DMA,
        pltpu.SemaphoreType.DMA,
    ],
)
def _sc_gather(logp_hbm, words_hbm, out_hbm,
               idx_v, buf0, buf1, gsem0, gsem1, ssem0, ssem1):
    wid = lax.axis_index("s") * NC + lax.axis_index("c")
    row0 = wid * B_PER_W
    bufs = (buf0, buf1)
    gsems = (gsem0, gsem1)
    ssems = (ssem0, ssem1)

    pltpu.sync_copy(words_hbm.at[wid], idx_v)

    def fire_gather(j, b):
        pltpu.async_copy(logp_hbm.at[idx_v.at[j]], bufs[b], gsems[b])

    def wait_gather(b):
        pltpu.make_async_copy(logp_hbm.at[pl.ds(0, CH)], bufs[b],
                              gsems[b]).wait()

    def fire_scatter(j, b):
        pltpu.async_copy(bufs[b], out_hbm.at[pl.ds(row0 + j * CH, CH)],
                         ssems[b])

    def wait_scatter(b):
        pltpu.make_async_copy(bufs[b], out_hbm.at[pl.ds(0, CH)],
                              ssems[b]).wait()

    fire_gather(0, 0)

    def pair(jo, carry):
        for b in range(2):
            j = jo * 2 + b
            nxt = j + 1

            @pl.when(nxt < NCH)
            def _fire_next():
                @pl.when(nxt >= 2)
                def _recycle():
                    wait_scatter(1 - b)
                fire_gather(nxt, 1 - b)

            wait_gather(b)
            fire_scatter(j, b)
        return carry

    lax.fori_loop(0, NCH // 2, pair, 0)
    wait_scatter(0)
    wait_scatter(1)


def kernel(words, weight):
    negc = _compute_negc(weight)
    logp = _compute_logp(weight, negc)
    words3d = words.reshape(NW, NCH, CH)
    out = _sc_gather(logp, words3d)
    return out.reshape(words.shape + (N_LABELS,))
